# baseline (speedup 1.0000x reference)
import functools

import jax
import jax.numpy as jnp
from jax import lax
from jax.experimental import pallas as pl
from jax.experimental.pallas import tpu as pltpu
from jax.experimental.pallas import tpu_sc as plsc

_B = 16384
_DIM = 32
_NC = 2
_NS = 16
_NW = _NC * _NS
_BPW = _B // _NW
_CH = 128
_NCH = _BPW // _CH

_BLK = 2048


def _gather_body(user_hbm, item_hbm, gu_t, gi_t, mu_t, mi_t,
                 gu_o, gi_o, mu_o, mi_o,
                 uidx, iidx, gu_b, gi_b, mu_b, mi_b, sem):
    wid = lax.axis_index("s") * _NC + lax.axis_index("c")
    row0 = wid * _NCH
    base = wid * _BPW
    pltpu.sync_copy(user_hbm.at[pl.ds(row0, _NCH)], uidx)
    pltpu.sync_copy(item_hbm.at[pl.ds(row0, _NCH)], iidx)
    copies = []
    for j in range(_NCH):
        dst = pl.ds(j * _CH, _CH)
        copies.append(pltpu.async_copy(gu_t.at[uidx.at[j]], gu_b.at[dst], sem))
        copies.append(pltpu.async_copy(gi_t.at[iidx.at[j]], gi_b.at[dst], sem))
        copies.append(pltpu.async_copy(mu_t.at[uidx.at[j]], mu_b.at[dst], sem))
        copies.append(pltpu.async_copy(mi_t.at[iidx.at[j]], mi_b.at[dst], sem))
    for c in copies:
        c.wait()
    pltpu.sync_copy(gu_b, gu_o.at[pl.ds(base, _BPW)])
    pltpu.sync_copy(gi_b, gi_o.at[pl.ds(base, _BPW)])
    pltpu.sync_copy(mu_b, mu_o.at[pl.ds(base, _BPW)])
    pltpu.sync_copy(mi_b, mi_o.at[pl.ds(base, _BPW)])


@functools.lru_cache(maxsize=None)
def _make_gather():
    return pl.kernel(
        _gather_body,
        out_type=[jax.ShapeDtypeStruct((_B, _DIM), jnp.float32)] * 4,
        mesh=plsc.VectorSubcoreMesh(core_axis_name="c", subcore_axis_name="s"),
        compiler_params=pltpu.CompilerParams(use_tc_tiling_on_sc=False),
        scratch_types=[
            pltpu.VMEM((_NCH, _CH), jnp.int32),
            pltpu.VMEM((_NCH, _CH), jnp.int32),
            pltpu.VMEM((_BPW, _DIM), jnp.float32),
            pltpu.VMEM((_BPW, _DIM), jnp.float32),
            pltpu.VMEM((_BPW, _DIM), jnp.float32),
            pltpu.VMEM((_BPW, _DIM), jnp.float32),
            pltpu.SemaphoreType.DMA,
        ],
    )


def _mlp_body(gu, gi, mu, mi, w0u, w0m, b0, w1, b1, w2, b2, wfg, wfm, bf,
              out):
    h = jnp.maximum(mu[...] @ w0u[...] + mi[...] @ w0m[...] + b0[...], 0.0)
    h = jnp.maximum(h @ w1[...] + b1[...], 0.0)
    h = jnp.maximum(h @ w2[...] + b2[...], 0.0)
    g = gu[...] * gi[...]
    out[...] = (jnp.sum(g * wfg[...], axis=1)
                + jnp.sum(h * wfm[...], axis=1) + bf[0, 0])


def _full(shape):
    return pl.BlockSpec(shape, lambda i: (0,) * len(shape))


_mlp_head = pl.pallas_call(
    _mlp_body,
    grid=(_B // _BLK,),
    in_specs=[
        pl.BlockSpec((_BLK, _DIM), lambda i: (i, 0)),
        pl.BlockSpec((_BLK, _DIM), lambda i: (i, 0)),
        pl.BlockSpec((_BLK, _DIM), lambda i: (i, 0)),
        pl.BlockSpec((_BLK, _DIM), lambda i: (i, 0)),
        _full((_DIM, 64)),
        _full((_DIM, 64)),
        _full((1, 64)),
        _full((64, 32)),
        _full((1, 32)),
        _full((32, 16)),
        _full((1, 16)),
        _full((1, _DIM)),
        _full((1, 16)),
        _full((1, 1)),
    ],
    out_specs=pl.BlockSpec((_BLK,), lambda i: (i,)),
    out_shape=jax.ShapeDtypeStruct((_B,), jnp.float32),
)


def kernel(user, item, gmf_user_table, gmf_item_table, mlp_user_table,
           mlp_item_table, W0, b0, W1, b1, W2, b2, Wf, bf):
    u2 = user.astype(jnp.int32).reshape(_B // _CH, _CH)
    i2 = item.astype(jnp.int32).reshape(_B // _CH, _CH)
    gu, gi, mu, mi = _make_gather()(u2, i2, gmf_user_table, gmf_item_table,
                                    mlp_user_table, mlp_item_table)
    return _mlp_head(
        gu, gi, mu, mi,
        W0[:_DIM], W0[_DIM:], b0.reshape(1, 64),
        W1, b1.reshape(1, 32), W2, b2.reshape(1, 16),
        Wf[:_DIM].reshape(1, _DIM), Wf[_DIM:].reshape(1, 16),
        bf.reshape(1, 1))
